# baseline (speedup 1.0000x reference)
import functools

import jax
import jax.numpy as jnp
from jax import lax
from jax.experimental import pallas as pl
from jax.experimental.pallas import tpu as pltpu, tpu_sc as plsc

VOCAB = 30522
HIDDEN = 768
MAX_POS = 512
BATCH = 1024
SEQ = 512
EPS = 1e-12

L = 16
NSLICE = HIDDEN // L
NW = 32
SEQ_PER_W = BATCH // NW
C = 64
NCHUNK = SEQ // C


def _sc_body(ids_hbm, word_hbm, pos_hbm, type_hbm, gamma_hbm, beta_hbm,
             out_hbm, idx_v, rows_v, pos_v, type_v, gamma_v, beta_v,
             acc_v, acc2_v, mean_v, rstd_v, sem):
    wid = lax.axis_index("s") * 2 + lax.axis_index("c")

    pltpu.sync_copy(type_hbm.at[pl.ds(0, 1)], type_v)
    pltpu.sync_copy(gamma_hbm, gamma_v)
    pltpu.sync_copy(beta_hbm, beta_v)

    def chunk_body(ci, carry):
        s0 = ci * C
        pltpu.sync_copy(pos_hbm.at[pl.ds(s0, C)], pos_v)

        def add_type(t, c):
            for j in range(NSLICE):
                sl = pl.ds(j * L, L)
                pos_v[t, sl] = pos_v[t, sl] + type_v[0, sl]
            return c

        lax.fori_loop(0, C, add_type, 0)

        def seq_body(bl, c):
            base = (wid * SEQ_PER_W + bl) * SEQ + s0
            pltpu.sync_copy(ids_hbm.at[pl.ds(base, C)], idx_v)
            pltpu.async_copy(word_hbm.at[idx_v], rows_v, sem).wait()

            def tok_sums(t, cc):
                acc = jnp.zeros((L,), jnp.float32)
                acc2 = jnp.zeros((L,), jnp.float32)
                for j in range(NSLICE):
                    sl = pl.ds(j * L, L)
                    v = rows_v[t, sl] + pos_v[t, sl]
                    rows_v[t, sl] = v
                    acc = acc + v
                    acc2 = acc2 + v * v
                acc_v[t, :] = acc
                acc2_v[t, :] = acc2
                return cc

            lax.fori_loop(0, C, tok_sums, 0)

            def grp_stats(g, cc):
                toks = g * L + lax.iota(jnp.int32, L)
                s1 = jnp.zeros((L,), jnp.float32)
                s2 = jnp.zeros((L,), jnp.float32)
                for l in range(L):
                    lane = jnp.full((L,), l, jnp.int32)
                    s1 = s1 + plsc.load_gather(acc_v, [toks, lane])
                    s2 = s2 + plsc.load_gather(acc2_v, [toks, lane])
                mean = s1 * (1.0 / HIDDEN)
                x = jnp.maximum(s2 * (1.0 / HIDDEN) - mean * mean, 0.0) + EPS
                bits = plsc.bitcast(x, jnp.int32)
                y = plsc.bitcast(jnp.int32(0x5F3759DF) - (bits >> 1),
                                 jnp.float32)
                xh = x * 0.5
                for _ in range(3):
                    y = y * (1.5 - xh * y * y)
                mean_v[pl.ds(g * L, L)] = mean
                rstd_v[pl.ds(g * L, L)] = y
                return cc

            lax.fori_loop(0, C // L, grp_stats, 0)

            def tok_norm(t, cc):
                tt = jnp.full((L,), t, jnp.int32)
                mv = plsc.load_gather(mean_v, [tt])
                rs = plsc.load_gather(rstd_v, [tt])
                for j in range(NSLICE):
                    sl = pl.ds(j * L, L)
                    v = (rows_v[t, sl] - mv) * rs
                    rows_v[t, sl] = v * gamma_v[sl] + beta_v[sl]
                return cc

            lax.fori_loop(0, C, tok_norm, 0)
            pltpu.sync_copy(rows_v, out_hbm.at[pl.ds(base, C)])
            return c

        lax.fori_loop(0, SEQ_PER_W, seq_body, 0)
        return carry

    lax.fori_loop(0, NCHUNK, chunk_body, 0)


@jax.jit
def kernel(input_ids, word_emb, pos_emb, type_emb, gamma, beta):
    ids_flat = input_ids.reshape(-1).astype(jnp.int32)
    mesh = plsc.VectorSubcoreMesh(core_axis_name="c", subcore_axis_name="s")
    k = functools.partial(
        pl.kernel,
        out_type=jax.ShapeDtypeStruct((BATCH * SEQ, HIDDEN), jnp.float32),
        mesh=mesh,
        compiler_params=pltpu.CompilerParams(needs_layout_passes=False),
        scratch_types=[
            pltpu.VMEM((C,), jnp.int32),
            pltpu.VMEM((C, HIDDEN), jnp.float32),
            pltpu.VMEM((C, HIDDEN), jnp.float32),
            pltpu.VMEM((1, HIDDEN), jnp.float32),
            pltpu.VMEM((HIDDEN,), jnp.float32),
            pltpu.VMEM((HIDDEN,), jnp.float32),
            pltpu.VMEM((C, L), jnp.float32),
            pltpu.VMEM((C, L), jnp.float32),
            pltpu.VMEM((C,), jnp.float32),
            pltpu.VMEM((C,), jnp.float32),
            pltpu.SemaphoreType.DMA,
        ],
    )(_sc_body)
    out = k(ids_flat, word_emb, pos_emb, type_emb, gamma, beta)
    return out.reshape(BATCH, SEQ, HIDDEN)


# speedup vs baseline: 1.1741x; 1.1741x over previous
import functools

import jax
import jax.numpy as jnp
from jax import lax
from jax.experimental import pallas as pl
from jax.experimental.pallas import tpu as pltpu, tpu_sc as plsc

VOCAB = 30522
HIDDEN = 768
MAX_POS = 512
BATCH = 1024
SEQ = 512
EPS = 1e-12

L = 16
NSLICE = HIDDEN // L
NW = 32
SEQ_PER_W = BATCH // NW
C = 32
NCHUNK = SEQ // C
NPAIR = SEQ_PER_W // 2


def _sc_body(ids_hbm, word_hbm, pos_hbm, type_hbm, gamma_hbm, beta_hbm,
             out_hbm, idx_all, rows0, rows1, ob0, ob1, pos_v, type_v,
             gamma_v, beta_v, acc_v, acc2_v, mean_v, rstd_v,
             sg0, sg1, ss0, ss1, si):
    wid = lax.axis_index("s") * 2 + lax.axis_index("c")
    row0 = wid * SEQ_PER_W
    rows = (rows0, rows1)
    obs = (ob0, ob1)
    sgs = (sg0, sg1)
    sss = (ss0, ss1)

    pltpu.sync_copy(type_hbm.at[pl.ds(0, HIDDEN)], type_v)
    pltpu.sync_copy(gamma_hbm, gamma_v)
    pltpu.sync_copy(beta_hbm, beta_v)

    def pass1(rbuf, obuf):
        def tok(t, c):
            a = [jnp.zeros((L,), jnp.float32) for _ in range(4)]
            q = [jnp.zeros((L,), jnp.float32) for _ in range(4)]
            for j in range(NSLICE):
                sl = pl.ds(j * L, L)
                v = rbuf[t, sl] + pos_v[t, sl]
                obuf[t, sl] = v
                a[j % 4] = a[j % 4] + v
                q[j % 4] = q[j % 4] + v * v
            acc_v[pl.ds(t * L, L)] = (a[0] + a[1]) + (a[2] + a[3])
            acc2_v[pl.ds(t * L, L)] = (q[0] + q[1]) + (q[2] + q[3])
            return c

        lax.fori_loop(0, C, tok, 0)

    def stats():
        def grp(g, c):
            toks = g * L + lax.iota(jnp.int32, L)
            s1 = jnp.zeros((L,), jnp.float32)
            s2 = jnp.zeros((L,), jnp.float32)
            base16 = toks * L
            for l in range(L):
                s1 = s1 + plsc.load_gather(acc_v, [base16 + l])
                s2 = s2 + plsc.load_gather(acc2_v, [base16 + l])
            mean = s1 * (1.0 / HIDDEN)
            x = jnp.maximum(s2 * (1.0 / HIDDEN) - mean * mean, 0.0) + EPS
            bits = plsc.bitcast(x, jnp.int32)
            y = plsc.bitcast(jnp.int32(0x5F3759DF) - (bits >> 1), jnp.float32)
            xh = x * 0.5
            for _ in range(3):
                y = y * (1.5 - xh * y * y)
            mean_v[pl.ds(g * L, L)] = mean
            rstd_v[pl.ds(g * L, L)] = y
            return c

        lax.fori_loop(0, C // L, grp, 0)

    def pass2(obuf):
        def tok(t, c):
            tt = jnp.full((L,), t, jnp.int32)
            mv = plsc.load_gather(mean_v, [tt])
            rs = plsc.load_gather(rstd_v, [tt])
            for j in range(NSLICE):
                sl = pl.ds(j * L, L)
                v = (obuf[t, sl] - mv) * rs
                obuf[t, sl] = v * gamma_v[sl] + beta_v[sl]
            return c

        lax.fori_loop(0, C, tok, 0)

    def pair_body(p, carry):
        ci = p // NPAIR
        pi = p % NPAIR
        s0 = ci * C

        @pl.when(pi == 0)
        def _chunk_top():
            pltpu.sync_copy(pos_hbm.at[pl.ds(s0, C)], pos_v)

            def add_type(t, c):
                for j in range(NSLICE):
                    sl = pl.ds(j * L, L)
                    pos_v[t, sl] = pos_v[t, sl] + type_v[sl]
                return c

            lax.fori_loop(0, C, add_type, 0)
            for i in range(SEQ_PER_W):
                pltpu.async_copy(
                    ids_hbm.at[pl.ds((row0 + i) * SEQ + s0, C)],
                    idx_all.at[pl.ds(i * C, C)], si)
            for i in range(SEQ_PER_W):
                pltpu.make_async_copy(
                    ids_hbm.at[pl.ds((row0 + i) * SEQ + s0, C)],
                    idx_all.at[pl.ds(i * C, C)], si).wait()
            pltpu.async_copy(word_hbm.at[idx_all.at[pl.ds(0, C)]], rows0, sg0)

        for b in (0, 1):
            i = 2 * pi + b
            rbuf, obuf, sg, ss = rows[b], obs[b], sgs[b], sss[b]

            not_first = jnp.logical_or(ci > 0, i >= 2)

            @pl.when(not_first)
            def _wait_store(obuf=obuf, ss=ss):
                pltpu.make_async_copy(obuf, out_hbm.at[pl.ds(0, C)],
                                      ss).wait()

            @pl.when(i + 1 < SEQ_PER_W)
            def _issue_gather(b=b, i=i):
                pltpu.async_copy(word_hbm.at[idx_all.at[pl.ds((i + 1) * C, C)]],
                                 rows[1 - b], sgs[1 - b])

            pltpu.make_async_copy(word_hbm.at[idx_all.at[pl.ds(i * C, C)]], rbuf,
                                  sg).wait()
            pass1(rbuf, obuf)
            stats()
            pass2(obuf)
            base = (row0 + i) * SEQ + s0
            pltpu.async_copy(obuf, out_hbm.at[pl.ds(base, C)], ss)
        return carry

    lax.fori_loop(0, NCHUNK * NPAIR, pair_body, 0)
    pltpu.make_async_copy(ob0, out_hbm.at[pl.ds(0, C)], ss0).wait()
    pltpu.make_async_copy(ob1, out_hbm.at[pl.ds(0, C)], ss1).wait()


@jax.jit
def kernel(input_ids, word_emb, pos_emb, type_emb, gamma, beta):
    mesh = plsc.VectorSubcoreMesh(core_axis_name="c", subcore_axis_name="s")
    k = functools.partial(
        pl.kernel,
        out_type=jax.ShapeDtypeStruct((BATCH * SEQ, HIDDEN), jnp.float32),
        mesh=mesh,
        compiler_params=pltpu.CompilerParams(needs_layout_passes=False),
        scratch_types=[
            pltpu.VMEM((SEQ_PER_W * C,), jnp.int32),
            pltpu.VMEM((C, HIDDEN), jnp.float32),
            pltpu.VMEM((C, HIDDEN), jnp.float32),
            pltpu.VMEM((C, HIDDEN), jnp.float32),
            pltpu.VMEM((C, HIDDEN), jnp.float32),
            pltpu.VMEM((C, HIDDEN), jnp.float32),
            pltpu.VMEM((HIDDEN,), jnp.float32),
            pltpu.VMEM((HIDDEN,), jnp.float32),
            pltpu.VMEM((HIDDEN,), jnp.float32),
            pltpu.VMEM((C * L,), jnp.float32),
            pltpu.VMEM((C * L,), jnp.float32),
            pltpu.VMEM((C,), jnp.float32),
            pltpu.VMEM((C,), jnp.float32),
            pltpu.SemaphoreType.DMA,
            pltpu.SemaphoreType.DMA,
            pltpu.SemaphoreType.DMA,
            pltpu.SemaphoreType.DMA,
            pltpu.SemaphoreType.DMA,
        ],
    )(_sc_body)
    out = k(input_ids.reshape(-1).astype(jnp.int32), word_emb, pos_emb,
            type_emb.reshape(-1), gamma, beta)
    return out.reshape(BATCH, SEQ, HIDDEN)
